# initial kernel scaffold (unmeasured)
import jax
import jax.numpy as jnp
from jax import lax
from jax.experimental import pallas as pl
from jax.experimental.pallas import tpu as pltpu

N_DEV = 4
SQ = 2048
SKV = 2048
HQ_LOCAL = 8
DH = 128
D_MODEL = 1024
D_HEADS_LOCAL = HQ_LOCAL * DH
BLK = 64
Q_TILE = 512
SCALE = 0.08838834764831843


def kernel(x, Wq, K_ext, V_ext, Wo):
    my = lax.axis_index("i")
    x2 = x.reshape(SQ, D_MODEL)
    k2 = K_ext.reshape(SKV, D_HEADS_LOCAL)
    v2 = V_ext.reshape(SKV, D_HEADS_LOCAL)
    wq_s = lax.dynamic_slice_in_dim(Wq, my * D_HEADS_LOCAL, D_HEADS_LOCAL, axis=1)
    wo_s = lax.dynamic_slice_in_dim(Wo, my * D_HEADS_LOCAL, D_HEADS_LOCAL, axis=0)

    def body(x_ref, wq_ref, k_ref, v_ref, wo_ref, out_ref,
             ctx_ref, comm_ref, send_sems, recv_sems):
        my_pos = lax.axis_index("i")
        left = lax.rem(my_pos + N_DEV - 1, N_DEV)
        right = lax.rem(my_pos + 1, N_DEV)

        barrier_sem = pltpu.get_barrier_semaphore()
        for nbr in (left, right):
            pl.semaphore_signal(
                barrier_sem, inc=1,
                device_id=(nbr,), device_id_type=pl.DeviceIdType.MESH,
            )
        pl.semaphore_wait(barrier_sem, 2)

        q = jnp.dot(x_ref[...], wq_ref[...], preferred_element_type=jnp.float32)

        for h in range(HQ_LOCAL):
            c0, c1 = h * DH, (h + 1) * DH
            k_h = k_ref[:, c0:c1]
            v_h = v_ref[:, c0:c1]
            for t in range(SQ // Q_TILE):
                q0 = t * Q_TILE
                kl = q0 + Q_TILE
                q_t = q[q0:q0 + Q_TILE, c0:c1]
                s = lax.dot_general(
                    q_t, k_h[:kl, :],
                    (((1,), (1,)), ((), ())),
                    preferred_element_type=jnp.float32,
                ) * SCALE
                qb = (q0 + lax.broadcasted_iota(jnp.int32, (Q_TILE, kl), 0)) // BLK
                kb = lax.broadcasted_iota(jnp.int32, (Q_TILE, kl), 1) // BLK
                s = jnp.where(kb <= qb, s, -1e9)
                m = jnp.max(s, axis=1, keepdims=True)
                w = jnp.exp(s - m)
                w = w / jnp.sum(w, axis=1, keepdims=True)
                ctx_ref[q0:q0 + Q_TILE, c0:c1] = jnp.dot(
                    w, v_h[:kl, :], preferred_element_type=jnp.float32)

        partial = jnp.dot(ctx_ref[...], wo_ref[...],
                          preferred_element_type=jnp.float32)
        out_ref[...] = partial
        comm_ref[0] = partial

        for h in range(N_DEV - 1):
            rdma = pltpu.make_async_remote_copy(
                src_ref=comm_ref.at[h],
                dst_ref=comm_ref.at[h + 1],
                send_sem=send_sems.at[h],
                recv_sem=recv_sems.at[h],
                device_id=(right,),
                device_id_type=pl.DeviceIdType.MESH,
            )
            rdma.start()
            rdma.wait()
            out_ref[...] += comm_ref[h + 1]

    out = pl.pallas_call(
        body,
        out_shape=jax.ShapeDtypeStruct((SQ, D_MODEL), jnp.float32),
        in_specs=[pl.BlockSpec(memory_space=pltpu.VMEM)] * 5,
        out_specs=pl.BlockSpec(memory_space=pltpu.VMEM),
        scratch_shapes=[
            pltpu.VMEM((SQ, D_HEADS_LOCAL), jnp.float32),
            pltpu.VMEM((N_DEV, SQ, D_MODEL), jnp.float32),
            pltpu.SemaphoreType.DMA((N_DEV - 1,)),
            pltpu.SemaphoreType.DMA((N_DEV - 1,)),
        ],
        compiler_params=pltpu.CompilerParams(collective_id=0),
    )(x2, wq_s, k2, v2, wo_s)
    return out.reshape(1, SQ, D_MODEL)


# baseline (device time: 307276 ns/iter reference)
import jax
import jax.numpy as jnp
from jax import lax
from jax.experimental import pallas as pl
from jax.experimental.pallas import tpu as pltpu

N_DEV = 4
SQ = 2048
SKV = 2048
HQ_LOCAL = 8
DH = 128
D_MODEL = 1024
D_HEADS_LOCAL = HQ_LOCAL * DH
BLK = 64
Q_TILE = 256
N_TILES = SQ // Q_TILE
CHUNK = SQ // N_DEV
SCALE = 0.08838834764831843


def kernel(x, Wq, K_ext, V_ext, Wo):
    my = lax.axis_index("i")
    x2 = x.reshape(SQ, D_MODEL)
    k2 = K_ext.reshape(SKV, D_HEADS_LOCAL)
    v2 = V_ext.reshape(SKV, D_HEADS_LOCAL)
    wq_s = lax.dynamic_slice_in_dim(Wq, my * D_HEADS_LOCAL, D_HEADS_LOCAL, axis=1)
    wo_s = lax.dynamic_slice_in_dim(Wo, my * D_HEADS_LOCAL, D_HEADS_LOCAL, axis=0)

    def body(x_ref, wq_ref, k_ref, v_ref, wo_ref, out_ref,
             ctx_ref, kst_ref, vst_ref, rs_ref,
             kv_sems, send_sems, recv_sems):
        my_pos = lax.axis_index("i")
        left = lax.rem(my_pos + N_DEV - 1, N_DEV)
        right = lax.rem(my_pos + 1, N_DEV)

        barrier_sem = pltpu.get_barrier_semaphore()
        for nbr in (left, right):
            pl.semaphore_signal(
                barrier_sem, inc=1,
                device_id=(nbr,), device_id_type=pl.DeviceIdType.MESH,
            )
        pl.semaphore_wait(barrier_sem, 2)

        for t in range(N_TILES):
            q0 = t * Q_TILE
            kl = q0 + Q_TILE
            q_t = jnp.dot(x_ref[q0:q0 + Q_TILE, :], wq_ref[...],
                          preferred_element_type=jnp.float32)
            for h in range(HQ_LOCAL):
                c0, c1 = h * DH, (h + 1) * DH
                kcp = pltpu.make_async_copy(
                    k_ref.at[pl.ds(0, kl), c0:c1], kst_ref.at[pl.ds(0, kl)],
                    kv_sems.at[0])
                vcp = pltpu.make_async_copy(
                    v_ref.at[pl.ds(0, kl), c0:c1], vst_ref.at[pl.ds(0, kl)],
                    kv_sems.at[1])
                kcp.start()
                vcp.start()
                kcp.wait()
                vcp.wait()
                s = lax.dot_general(
                    q_t[:, c0:c1], kst_ref[:kl, :],
                    (((1,), (1,)), ((), ())),
                    preferred_element_type=jnp.float32,
                ) * SCALE
                qb = (q0 + lax.broadcasted_iota(jnp.int32, (Q_TILE, kl), 0)) // BLK
                kb = lax.broadcasted_iota(jnp.int32, (Q_TILE, kl), 1) // BLK
                s = jnp.where(kb <= qb, s, -1e9)
                m = jnp.max(s, axis=1, keepdims=True)
                w = jnp.exp(s - m)
                w = w / jnp.sum(w, axis=1, keepdims=True)
                ctx_ref[:, c0:c1] = jnp.dot(
                    w, vst_ref[:kl, :], preferred_element_type=jnp.float32)
            out_ref[q0:q0 + Q_TILE, :] = jnp.dot(
                ctx_ref[...], wo_ref[...], preferred_element_type=jnp.float32)

        for s in range(N_DEV - 1):
            src = (out_ref.at[pl.ds(my_pos * CHUNK, CHUNK), :] if s == 0
                   else rs_ref.at[s - 1])
            rdma = pltpu.make_async_remote_copy(
                src_ref=src,
                dst_ref=rs_ref.at[s],
                send_sem=send_sems.at[s],
                recv_sem=recv_sems.at[s],
                device_id=(right,),
                device_id_type=pl.DeviceIdType.MESH,
            )
            rdma.start()
            rdma.wait()
            c_recv = lax.rem(my_pos + 2 * N_DEV - s - 1, N_DEV)
            rs_ref[s] += out_ref[pl.ds(c_recv * CHUNK, CHUNK), :]

        c_own = lax.rem(my_pos + 1, N_DEV)
        out_ref[pl.ds(c_own * CHUNK, CHUNK), :] = rs_ref[N_DEV - 2]

        for s in range(N_DEV - 1):
            c_id = lax.rem(my_pos + 1 + N_DEV - s, N_DEV)
            rdma = pltpu.make_async_remote_copy(
                src_ref=out_ref.at[pl.ds(c_id * CHUNK, CHUNK), :],
                dst_ref=out_ref.at[pl.ds(c_id * CHUNK, CHUNK), :],
                send_sem=send_sems.at[N_DEV - 1 + s],
                recv_sem=recv_sems.at[N_DEV - 1 + s],
                device_id=(right,),
                device_id_type=pl.DeviceIdType.MESH,
            )
            rdma.start()
            rdma.wait()

    out = pl.pallas_call(
        body,
        out_shape=jax.ShapeDtypeStruct((SQ, D_MODEL), jnp.float32),
        in_specs=[
            pl.BlockSpec(memory_space=pltpu.VMEM),
            pl.BlockSpec(memory_space=pltpu.VMEM),
            pl.BlockSpec(memory_space=pltpu.MemorySpace.HBM),
            pl.BlockSpec(memory_space=pltpu.MemorySpace.HBM),
            pl.BlockSpec(memory_space=pltpu.VMEM),
        ],
        out_specs=pl.BlockSpec(memory_space=pltpu.VMEM),
        scratch_shapes=[
            pltpu.VMEM((Q_TILE, D_HEADS_LOCAL), jnp.float32),
            pltpu.VMEM((SKV, DH), jnp.float32),
            pltpu.VMEM((SKV, DH), jnp.float32),
            pltpu.VMEM((N_DEV - 1, CHUNK, D_MODEL), jnp.float32),
            pltpu.SemaphoreType.DMA((2,)),
            pltpu.SemaphoreType.DMA((2 * (N_DEV - 1),)),
            pltpu.SemaphoreType.DMA((2 * (N_DEV - 1),)),
        ],
        compiler_params=pltpu.CompilerParams(collective_id=0),
    )(x2, wq_s, k2, v2, wo_s)
    return out.reshape(1, SQ, D_MODEL)


# device time: 195243 ns/iter; 1.5738x vs baseline; 1.5738x over previous
import jax
import jax.numpy as jnp
from jax import lax
from jax.experimental import pallas as pl
from jax.experimental.pallas import tpu as pltpu

N_DEV = 4
SQ = 2048
SKV = 2048
HQ_LOCAL = 8
DH = 128
D_MODEL = 1024
D_HEADS_LOCAL = HQ_LOCAL * DH
BLK = 64
Q_TILE = 512
N_TILES = SQ // Q_TILE
CHUNK = SQ // N_DEV
HALF = D_MODEL // 2
SCALE = 0.08838834764831843


def kernel(x, Wq, K_ext, V_ext, Wo):
    my = lax.axis_index("i")
    x2 = x.reshape(SQ, D_MODEL)
    k2 = K_ext.reshape(SKV, D_HEADS_LOCAL)
    v2 = V_ext.reshape(SKV, D_HEADS_LOCAL)
    wq_s = lax.dynamic_slice_in_dim(Wq, my * D_HEADS_LOCAL, D_HEADS_LOCAL, axis=1)
    wo_s = lax.dynamic_slice_in_dim(Wo, my * D_HEADS_LOCAL, D_HEADS_LOCAL, axis=0)

    def body(x_ref, wq_ref, k_ref, v_ref, wo_ref, out_ref,
             kst_ref, vst_ref, rs_ref,
             kv_sems, send_sems, recv_sems):
        my_pos = lax.axis_index("i")
        left = lax.rem(my_pos + N_DEV - 1, N_DEV)
        right = lax.rem(my_pos + 1, N_DEV)

        barrier_sem = pltpu.get_barrier_semaphore()
        for nbr in (left, right):
            pl.semaphore_signal(
                barrier_sem, inc=1,
                device_id=(nbr,), device_id_type=pl.DeviceIdType.MESH,
            )
        pl.semaphore_wait(barrier_sem, 2)

        def start_kv(h):
            slot = h % 2
            c0, c1 = h * DH, (h + 1) * DH
            kcp = pltpu.make_async_copy(
                k_ref.at[:, c0:c1], kst_ref.at[slot], kv_sems.at[slot, 0])
            vcp = pltpu.make_async_copy(
                v_ref.at[:, c0:c1], vst_ref.at[slot], kv_sems.at[slot, 1])
            kcp.start()
            vcp.start()
            return kcp, vcp

        pending = start_kv(0)
        for h in range(HQ_LOCAL):
            c0, c1 = h * DH, (h + 1) * DH
            slot = h % 2
            q_h = jnp.dot(x_ref[...], wq_ref[:, c0:c1],
                          preferred_element_type=jnp.float32)
            pending[0].wait()
            pending[1].wait()
            if h + 1 < HQ_LOCAL:
                pending = start_kv(h + 1)
            for t in range(N_TILES):
                q0 = t * Q_TILE
                kl = q0 + Q_TILE
                s = lax.dot_general(
                    q_h[q0:q0 + Q_TILE, :], kst_ref[slot, :kl, :],
                    (((1,), (1,)), ((), ())),
                    preferred_element_type=jnp.float32,
                ) * SCALE
                qb = (q0 + lax.broadcasted_iota(jnp.int32, (Q_TILE, kl), 0)) // BLK
                kb = lax.broadcasted_iota(jnp.int32, (Q_TILE, kl), 1) // BLK
                s = jnp.where(kb <= qb, s, -1e9)
                m = jnp.max(s, axis=1, keepdims=True)
                w = jnp.exp(s - m)
                w = w / jnp.sum(w, axis=1, keepdims=True)
                ctx_t = jnp.dot(w, vst_ref[slot, :kl, :],
                                preferred_element_type=jnp.float32)
                proj = jnp.dot(ctx_t, wo_ref[c0:c1, :],
                               preferred_element_type=jnp.float32)
                if h == 0:
                    out_ref[q0:q0 + Q_TILE, :] = proj
                else:
                    out_ref[q0:q0 + Q_TILE, :] += proj

        P = 2 * N_DEV

        def col(d):
            return slice(0, HALF) if d == 0 else slice(HALF, D_MODEL)

        def rows(c):
            return pl.ds(c * CHUNK, CHUNK)

        def rs_chunk_send(d, s):
            return lax.rem(my_pos + (P - s if d == 0 else s), N_DEV)

        def rs_chunk_recv(d, s):
            return lax.rem(my_pos + (P - s - 1 if d == 0 else s + 1), N_DEV)

        def ag_chunk(d, s):
            return lax.rem(my_pos + (P + 1 - s if d == 0 else P - 1 + s), N_DEV)

        def nbr_out(d):
            return right if d == 0 else left

        for s in range(N_DEV - 1):
            rdmas = []
            for d in (0, 1):
                src = (out_ref.at[rows(rs_chunk_send(d, s)), col(d)] if s == 0
                       else rs_ref.at[2 * (s - 1) + d])
                rdma = pltpu.make_async_remote_copy(
                    src_ref=src,
                    dst_ref=rs_ref.at[2 * s + d],
                    send_sem=send_sems.at[2 * s + d],
                    recv_sem=recv_sems.at[2 * s + d],
                    device_id=(nbr_out(d),),
                    device_id_type=pl.DeviceIdType.MESH,
                )
                rdma.start()
                rdmas.append(rdma)
            for d in (0, 1):
                rdmas[d].wait()
                c_recv = rs_chunk_recv(d, s)
                rs_ref[2 * s + d] += out_ref[rows(c_recv), col(d)]

        for d in (0, 1):
            c_own = lax.rem(my_pos + (1 if d == 0 else N_DEV - 1), N_DEV)
            out_ref[rows(c_own), col(d)] = rs_ref[2 * (N_DEV - 2) + d]

        for s in range(N_DEV - 1):
            rdmas = []
            for d in (0, 1):
                c_id = ag_chunk(d, s)
                rdma = pltpu.make_async_remote_copy(
                    src_ref=out_ref.at[rows(c_id), col(d)],
                    dst_ref=out_ref.at[rows(c_id), col(d)],
                    send_sem=send_sems.at[2 * (N_DEV - 1) + 2 * s + d],
                    recv_sem=recv_sems.at[2 * (N_DEV - 1) + 2 * s + d],
                    device_id=(nbr_out(d),),
                    device_id_type=pl.DeviceIdType.MESH,
                )
                rdma.start()
                rdmas.append(rdma)
            for d in (0, 1):
                rdmas[d].wait()

    out = pl.pallas_call(
        body,
        out_shape=jax.ShapeDtypeStruct((SQ, D_MODEL), jnp.float32),
        in_specs=[
            pl.BlockSpec(memory_space=pltpu.VMEM),
            pl.BlockSpec(memory_space=pltpu.VMEM),
            pl.BlockSpec(memory_space=pltpu.MemorySpace.HBM),
            pl.BlockSpec(memory_space=pltpu.MemorySpace.HBM),
            pl.BlockSpec(memory_space=pltpu.VMEM),
        ],
        out_specs=pl.BlockSpec(memory_space=pltpu.VMEM),
        scratch_shapes=[
            pltpu.VMEM((2, SKV, DH), jnp.float32),
            pltpu.VMEM((2, SKV, DH), jnp.float32),
            pltpu.VMEM((2 * (N_DEV - 1), CHUNK, HALF), jnp.float32),
            pltpu.SemaphoreType.DMA((2, 2)),
            pltpu.SemaphoreType.DMA((4 * (N_DEV - 1),)),
            pltpu.SemaphoreType.DMA((4 * (N_DEV - 1),)),
        ],
        compiler_params=pltpu.CompilerParams(collective_id=0),
    )(x2, wq_s, k2, v2, wo_s)
    return out.reshape(1, SQ, D_MODEL)
